# baseline (speedup 1.0000x reference)
import functools

import jax
import jax.numpy as jnp
from jax import lax
from jax.experimental import pallas as pl
from jax.experimental.pallas import tpu as pltpu
from jax.experimental.pallas import tpu_sc as plsc

_CHANNELS = 256
_GRID = 32
_N = 20000
_NW = 32
_BPW = 640
_CHUNK = 128
_NCHUNK = _BPW // _CHUNK
_MAGIC = 12582912.0
_MAGIC32 = 402653184.0
_IDXW = 136


@functools.partial(
    pl.kernel,
    mesh=plsc.VectorSubcoreMesh(core_axis_name="c", subcore_axis_name="s"),
    out_type=jax.ShapeDtypeStruct((_N, _CHANNELS), jnp.float32),
    scratch_types=[
        pltpu.VMEM((2 * _BPW,), jnp.float32),
        pltpu.VMEM((2 * _BPW,), jnp.float32),
        pltpu.VMEM((264,), jnp.float32),
        pltpu.VMEM((_NCHUNK * _IDXW,), jnp.int32),
        pltpu.VMEM((_CHUNK, _CHANNELS), jnp.float32),
        pltpu.VMEM((_CHUNK, _CHANNELS), jnp.float32),
        pltpu.SemaphoreType.DMA,
        pltpu.SemaphoreType.DMA,
    ],
)
def _pos_enc_sc(xy1_hbm, xy2_hbm, sup_hbm, out_hbm, a_v, b_v, t_v, idx_v,
                buf0, buf1, sem0, sem1):
    bufs = (buf0, buf1)
    sems = (sem0, sem1)

    wid = lax.axis_index("s") * 2 + lax.axis_index("c")
    base = jnp.minimum(wid * _BPW, _N - _BPW)

    cp_a = pltpu.async_copy(xy1_hbm.at[pl.ds(2 * base, 2 * _BPW)], a_v, sem0)
    cp_b = pltpu.async_copy(xy2_hbm.at[pl.ds(2 * base, 2 * _BPW)], b_v, sem1)
    cp_a.wait()
    cp_b.wait()

    lanes = lax.iota(jnp.int32, 16)
    perm = (lanes * 2) % 16

    gathers = [None] * _NCHUNK
    for c in range(_NCHUNK):
        for j in range(16):
            o = (c * 16 + j) * 16
            t_v[pl.ds(j * 16, 16)] = a_v[pl.ds(o, 16)] + b_v[pl.ds(o, 16)]
        for j in range(16):
            e = t_v[pl.ds(j * 16, 16)]
            o = t_v[pl.ds(j * 16 + 1, 16)]
            vx = (e * 496.0 + _MAGIC32) - _MAGIC32
            vy = (o * 15.5 + _MAGIC) - _MAGIC
            q = (jnp.clip(vx, 0.0, 992.0)
                 + jnp.clip(vy, 0.0, 31.0)).astype(jnp.int32)
            qc = q.at[perm].get(mode="promise_in_bounds")
            idx_v[pl.ds(c * _IDXW + j * 8, 16)] = qc
        gathers[c] = pltpu.async_copy(
            sup_hbm.at[idx_v.at[pl.ds(c * _IDXW, _CHUNK)]], bufs[c % 2],
            sems[c % 2])
        if c >= 1:
            gathers[c - 1].wait()
            pltpu.sync_copy(bufs[(c - 1) % 2],
                            out_hbm.at[pl.ds(base + (c - 1) * _CHUNK, _CHUNK)])
    c = _NCHUNK - 1
    gathers[c].wait()
    pltpu.sync_copy(bufs[c % 2],
                    out_hbm.at[pl.ds(base + c * _CHUNK, _CHUNK)])


def kernel(boxes_norm, row_embed, col_embed):
    xy1 = boxes_norm[:, :2].reshape(-1)
    xy2 = boxes_norm[:, 2:].reshape(-1)
    tbl = jnp.zeros((2 * _GRID, _CHANNELS), jnp.float32)
    tbl = lax.dynamic_update_slice(tbl, col_embed, (0, 0))
    tbl = lax.dynamic_update_slice(tbl, row_embed, (_GRID, 128))
    i = jnp.arange(_GRID * _GRID)
    g = jnp.arange(2 * _GRID)
    twohot = ((i[:, None] // _GRID == g[None, :])
              | (_GRID + i[:, None] % _GRID == g[None, :])).astype(jnp.float32)
    sup = jax.lax.dot(twohot, tbl, precision=jax.lax.Precision.HIGHEST)
    out = _pos_enc_sc(xy1, xy2, sup)
    return out[:, :, None, None]


# speedup vs baseline: 1.4070x; 1.4070x over previous
import functools

import jax
import jax.numpy as jnp
from jax import lax
from jax.experimental import pallas as pl
from jax.experimental.pallas import tpu as pltpu
from jax.experimental.pallas import tpu_sc as plsc

_CHANNELS = 256
_GRID = 32
_N = 20000
_NW = 32
_BPW = 640
_CHUNK = 128
_NCHUNK = _BPW // _CHUNK
_MAGIC = 12582912.0
_MAGIC32 = 402653184.0


@functools.partial(
    pl.kernel,
    mesh=plsc.VectorSubcoreMesh(core_axis_name="c", subcore_axis_name="s"),
    out_type=jax.ShapeDtypeStruct((_N, _CHANNELS), jnp.float32),
    scratch_types=[
        pltpu.VMEM((4 * _BPW,), jnp.float32),
        pltpu.VMEM((_BPW,), jnp.int32),
        pltpu.VMEM((_CHUNK, _CHANNELS), jnp.float32),
        pltpu.VMEM((_CHUNK, _CHANNELS), jnp.float32),
        pltpu.SemaphoreType.DMA,
        pltpu.SemaphoreType.DMA,
        pltpu.SemaphoreType.DMA,
        pltpu.SemaphoreType.DMA,
    ],
)
def _pos_enc_sc(x1_hbm, y1_hbm, x2_hbm, y2_hbm, sup_hbm, out_hbm, coord_v,
                idx_v, buf0, buf1, sem0, sem1, sem2, sem3):
    bufs = (buf0, buf1)
    sems = (sem0, sem1)

    wid = lax.axis_index("s") * 2 + lax.axis_index("c")
    base = jnp.minimum(wid * _BPW, _N - _BPW)

    stage = []
    for i, src in enumerate((x1_hbm, y1_hbm, x2_hbm, y2_hbm)):
        stage.append(pltpu.async_copy(src.at[pl.ds(base, _BPW)],
                                      coord_v.at[pl.ds(i * _BPW, _BPW)],
                                      (sem0, sem1, sem2, sem3)[i]))
    for cp in stage:
        cp.wait()

    gathers = [None] * _NCHUNK
    for c in range(_NCHUNK):
        for j in range(8):
            o = (c * 8 + j) * 16
            sx = coord_v[pl.ds(o, 16)] + coord_v[pl.ds(2 * _BPW + o, 16)]
            sy = coord_v[pl.ds(_BPW + o, 16)] + coord_v[pl.ds(3 * _BPW + o, 16)]
            vx = (sx * 496.0 + _MAGIC32) - _MAGIC32
            vy = (sy * 15.5 + _MAGIC) - _MAGIC
            idx = (jnp.clip(vx, 0.0, 992.0)
                   + jnp.clip(vy, 0.0, 31.0)).astype(jnp.int32)
            idx_v[pl.ds(c * _CHUNK + j * 16, 16)] = idx
        gathers[c] = pltpu.async_copy(
            sup_hbm.at[idx_v.at[pl.ds(c * _CHUNK, _CHUNK)]], bufs[c % 2],
            sems[c % 2])
        if c >= 1:
            gathers[c - 1].wait()
            pltpu.sync_copy(bufs[(c - 1) % 2],
                            out_hbm.at[pl.ds(base + (c - 1) * _CHUNK, _CHUNK)])
    c = _NCHUNK - 1
    gathers[c].wait()
    pltpu.sync_copy(bufs[c % 2],
                    out_hbm.at[pl.ds(base + c * _CHUNK, _CHUNK)])


def kernel(boxes_norm, row_embed, col_embed):
    x1 = boxes_norm[:, 0]
    y1 = boxes_norm[:, 1]
    x2 = boxes_norm[:, 2]
    y2 = boxes_norm[:, 3]
    tbl = jnp.zeros((2 * _GRID, _CHANNELS), jnp.float32)
    tbl = lax.dynamic_update_slice(tbl, col_embed, (0, 0))
    tbl = lax.dynamic_update_slice(tbl, row_embed, (_GRID, 128))
    i = jnp.arange(_GRID * _GRID)
    g = jnp.arange(2 * _GRID)
    twohot = ((i[:, None] // _GRID == g[None, :])
              | (_GRID + i[:, None] % _GRID == g[None, :])).astype(jnp.float32)
    sup = jax.lax.dot(twohot, tbl, precision=jax.lax.Precision.HIGHEST)
    out = _pos_enc_sc(x1, y1, x2, y2, sup)
    return out[:, :, None, None]
